# baseline (speedup 1.0000x reference)
import jax
import jax.numpy as jnp
import numpy as np
from jax.experimental import pallas as pl
from jax.experimental.pallas import tpu as pltpu
from jax.experimental.shard_map import shard_map
from jax.sharding import Mesh, PartitionSpec as P

C = 256
NCONV = 4
GROUPS = 32
EPS = 1e-5
LEVEL_HW = (64, 32, 16, 8, 4)

ACT_DT = jnp.bfloat16
MM_DT = jnp.bfloat16

GEOM = []
for _hw in LEVEL_HW:
    _Hp, _Wp = _hw + 2, _hw + 2
    _Np = _Hp * _Wp
    _S = _Wp + 1
    GEOM.append((_hw, _hw, _Hp, _Wp, _Np, _S, _Np + 2 * _S))


def _fcos_kernel(*refs, k):
    i = 0
    x_refs = refs[i:i + 5]; i += 5
    tw, bb, gg, be, hw_, hb, sv, rs, gmat = refs[i:i + 9]; i += 9
    m_refs = refs[i:i + 5]; i += 5
    o_refs = refs[i:i + 5]; i += 5
    scr = refs[i:i + 5 * k]

    gmat_v = gmat[...]

    units = []
    for ti in range(k):
        for l, (H, W, Hp, Wp, Np, S, T) in enumerate(GEOM):
            units.append(dict(
                ti=ti, l=l, X=scr[5 * ti + l], Np=Np, S=S, T=T,
                offs=[dy * Wp + dx for dy in (-1, 0, 1) for dx in (-1, 0, 1)],
                inv_cnt=1.0 / float(H * W * (C // GROUPS))))

    def conv9(u, wref, base):
        X, S, Np, ti = u['X'], u['S'], u['Np'], u['ti']
        acc = None
        for t, off in enumerate(u['offs']):
            xs = X[pl.ds(S + off, Np), :]
            p = jnp.dot(xs, wref[ti, base + t],
                        preferred_element_type=jnp.float32)
            acc = p if acc is None else acc + p
        return acc

    def gn_relu(u, acc, i):
        ti = u['ti']
        mask = m_refs[u['l']][...]
        ym = (acc + bb[ti, i]) * mask
        s1 = jnp.sum(ym, axis=0, keepdims=True)
        s2 = jnp.sum(ym * ym, axis=0, keepdims=True)
        gs1 = jnp.dot(s1, gmat_v, preferred_element_type=jnp.float32)
        gs2 = jnp.dot(s2, gmat_v, preferred_element_type=jnp.float32)
        mean = gs1 * u['inv_cnt']
        var = gs2 * u['inv_cnt'] - mean * mean
        a = jax.lax.rsqrt(var + EPS) * gg[ti, i]
        c = be[ti, i] - mean * a
        return (jnp.maximum(ym * a + c, 0.0) * mask).astype(ACT_DT)

    for u in units:
        u['X'][pl.ds(0, u['T']), :] = jnp.zeros((u['T'], C), ACT_DT)
        u['X'][pl.ds(u['S'], u['Np']), :] = x_refs[u['l']][...]

    pend = None
    for i in range(NCONV):
        for u in units:
            acc = conv9(u, tw, i * 9)
            if pend is not None:
                pu, pacc, pi = pend
                pu['X'][pl.ds(pu['S'], pu['Np']), :] = gn_relu(pu, pacc, pi)
            pend = (u, acc, i)
    pu, pacc, pi = pend
    pu['X'][pl.ds(pu['S'], pu['Np']), :] = gn_relu(pu, pacc, pi)

    for u in units:
        ti, l = u['ti'], u['l']
        v = (conv9(u, hw_, 0) + hb[ti]) * sv[ti, l]
        r = rs[ti]
        o_refs[l][ti] = (r * jnp.maximum(v, 0.0) + (1.0 - r) * v).astype(ACT_DT)


def _tower_taps(w):
    return w.transpose(0, 3, 4, 2, 1).reshape(NCONV * 9, C, C).astype(MM_DT)


def _head_taps(w, pad_to=128):
    o = w.shape[0]
    t = w.transpose(2, 3, 1, 0).reshape(9, C, o)
    return jnp.pad(t, ((0, 0), (0, 0), (0, pad_to - o))).astype(MM_DT)


def kernel(p3, p4, p5, p6, p7,
           cls_w, cls_b, cls_gn_g, cls_gn_b,
           bbox_w, bbox_b, bbox_gn_g, bbox_gn_b,
           head_cls_w, head_cls_b, head_bbox_w, head_bbox_b,
           head_ctr_w, head_ctr_b, scales):
    feats = (p3, p4, p5, p6, p7)
    xs, masks = [], []
    for (H, W, Hp, Wp, Np, S, T), f in zip(GEOM, feats):
        x = jnp.pad(f[0].transpose(1, 2, 0), ((1, 1), (1, 1), (0, 0)))
        xs.append(x.reshape(Np, C).astype(ACT_DT))
        m = np.zeros((Hp, Wp, 1), np.float32)
        m[1:H + 1, 1:W + 1] = 1.0
        masks.append(jnp.asarray(m.reshape(Np, 1)))

    tw = jnp.stack([_tower_taps(cls_w), _tower_taps(bbox_w)])
    per_layer = lambda a: a.reshape(NCONV, 1, C).astype(jnp.float32)
    bb = jnp.stack([per_layer(cls_b), per_layer(bbox_b)])
    gg = jnp.stack([per_layer(cls_gn_g), per_layer(bbox_gn_g)])
    be = jnp.stack([per_layer(cls_gn_b), per_layer(bbox_gn_b)])

    hw_ = jnp.stack([
        _head_taps(head_cls_w),
        _head_taps(jnp.concatenate([head_bbox_w, head_ctr_w], axis=0))])
    hb = jnp.stack([
        jnp.pad(head_cls_b, (0, 128 - 80)).reshape(1, 128).astype(jnp.float32),
        jnp.pad(jnp.concatenate([head_bbox_b, head_ctr_b]), (0, 128 - 5)
                ).reshape(1, 128).astype(jnp.float32)])

    lane = np.arange(128)
    sv_box = jnp.where(jnp.asarray(lane[None, None, :] < 4),
                       scales[:, None, None].astype(jnp.float32), 1.0)
    sv = jnp.stack([jnp.ones((5, 1, 128), jnp.float32), sv_box])
    rs = jnp.stack([jnp.zeros((1, 128), jnp.float32),
                    jnp.asarray((lane[None, :] < 4).astype(np.float32))])

    gmat = jnp.asarray(np.kron(np.eye(GROUPS, dtype=np.float32),
                               np.ones((C // GROUPS, C // GROUPS), np.float32)))

    devs = jax.devices()
    ndev = 2 if len(devs) >= 2 else 1
    mesh = Mesh(np.array(devs[:ndev]), ('t',))
    k = 2 // ndev

    def shard_body(tw, bb, gg, be, hw_, hb, sv, rs, gmat, xs, masks):
        out_shape = [jax.ShapeDtypeStruct((k, g[4], 128), ACT_DT) for g in GEOM]
        import functools
        return pl.pallas_call(
            functools.partial(_fcos_kernel, k=k),
            out_shape=out_shape,
            scratch_shapes=[pltpu.VMEM((g[6], C), ACT_DT)
                            for _ in range(k) for g in GEOM],
        )(*xs, tw, bb, gg, be, hw_, hb, sv, rs, gmat, *masks)

    outs = shard_map(
        shard_body, mesh=mesh,
        in_specs=(P('t'), P('t'), P('t'), P('t'), P('t'), P('t'), P('t'),
                  P('t'), P(), P(), P()),
        out_specs=P('t'), check_rep=False,
    )(tw, bb, gg, be, hw_, hb, sv, rs, gmat, tuple(xs), tuple(masks))

    logits, bboxs, ctrs = [], [], []
    for l, (H, W, Hp, Wp, Np, S, T) in enumerate(GEOM):
        o = outs[l].astype(jnp.float32).reshape(2, Hp, Wp, 128)
        lo = o[0, 1:H + 1, 1:W + 1, :80]
        logits.append(lo.transpose(2, 0, 1)[None])
        bc = o[1, 1:H + 1, 1:W + 1, :5]
        bboxs.append(bc[..., 0:4].transpose(2, 0, 1)[None])
        ctrs.append(bc[..., 4:5].transpose(2, 0, 1)[None])
    return tuple(logits) + tuple(bboxs) + tuple(ctrs)
